# Initial kernel scaffold; baseline (speedup 1.0000x reference)
import jax
import jax.numpy as jnp
from jax.experimental import pallas as pl


def kernel(x, S, w, b, thresh_t):
    raise NotImplementedError("write your pallas kernel here")


# speedup vs baseline: 226.7903x; 226.7903x over previous
import functools
import jax
import jax.numpy as jnp
from jax import lax
from jax.experimental import pallas as pl
from jax.experimental.pallas import tpu as pltpu
from jax.experimental.pallas import tpu_sc as plsc

B = 16384
L = 200
NCORES = 2
NSUB = 16
NW = NCORES * NSUB
ROWS_W = B // NW
CHUNK_ROWS = 64
NCHUNK = ROWS_W // CHUNK_ROWS
CHUNK_ELEMS = CHUNK_ROWS * L
VOCAB1 = 1000001
SPAD = 1024000
SLICE = SPAD // NSUB


def _sc_body(x_hbm, s_hbm, scal_hbm, logit_hbm, label_hbm,
             idx_v, val_v, out_f, out_i, scal_v, table_sp, sem):
    c = lax.axis_index("c")
    s = lax.axis_index("s")
    wid = s * NCORES + c

    for j in range(SLICE // CHUNK_ELEMS):
        o = s * SLICE + j * CHUNK_ELEMS
        pltpu.sync_copy(s_hbm.at[pl.ds(o, CHUNK_ELEMS)], val_v)
        pltpu.sync_copy(val_v, table_sp.at[pl.ds(o, CHUNK_ELEMS)])

    pltpu.sync_copy(scal_hbm, scal_v)
    plsc.subcore_barrier()

    sv = scal_v[...]
    w_s = sv[0]
    b_s = sv[1]
    t_s = sv[2]

    lane = jax.lax.iota(jnp.int32, 16)

    for k in range(NCHUNK):
        e0 = wid * ROWS_W * L + k * CHUNK_ELEMS
        pltpu.sync_copy(x_hbm.at[pl.ds(e0, CHUNK_ELEMS)], idx_v)
        pltpu.async_copy(table_sp.at[idx_v], val_v, sem).wait()

        for g in range(CHUNK_ROWS // 16):
            offs0 = (lane + g * 16) * L

            def body(l, carry):
                offs, acc = carry
                v = plsc.load_gather(val_v, [offs])
                return offs + 1, acc + v

            _, acc = lax.fori_loop(
                0, L, body, (offs0, jnp.zeros((16,), jnp.float32)))

            logit = acc * w_s + b_s
            lab = jnp.where(logit >= t_s,
                            jnp.full((16,), 1, jnp.int32),
                            jnp.full((16,), 0, jnp.int32))
            o = k * CHUNK_ROWS + g * 16
            out_f[pl.ds(o, 16)] = logit
            out_i[pl.ds(o, 16)] = lab

    pltpu.sync_copy(out_f, logit_hbm.at[pl.ds(wid * ROWS_W, ROWS_W)])
    pltpu.sync_copy(out_i, label_hbm.at[pl.ds(wid * ROWS_W, ROWS_W)])


@jax.jit
def _run(x2, s1, scal):
    f = functools.partial(
        pl.kernel,
        out_type=(jax.ShapeDtypeStruct((B,), jnp.float32),
                  jax.ShapeDtypeStruct((B,), jnp.int32)),
        mesh=plsc.VectorSubcoreMesh(core_axis_name="c", subcore_axis_name="s"),
        compiler_params=pltpu.CompilerParams(needs_layout_passes=False),
        scratch_types=[
            pltpu.VMEM((CHUNK_ELEMS,), jnp.int32),
            pltpu.VMEM((CHUNK_ELEMS,), jnp.float32),
            pltpu.VMEM((ROWS_W,), jnp.float32),
            pltpu.VMEM((ROWS_W,), jnp.int32),
            pltpu.VMEM((16,), jnp.float32),
            pltpu.VMEM_SHARED((SPAD,), jnp.float32),
            pltpu.SemaphoreType.DMA,
        ],
    )(_sc_body)
    return f(x2, s1, scal)


def kernel(x, S, w, b, thresh_t):
    x2 = x.reshape(-1)
    s1 = jnp.pad(S, ((0, SPAD - VOCAB1), (0, 0))).reshape(-1)
    scal = jnp.concatenate(
        [w, b, thresh_t.reshape(-1), jnp.zeros((13,), jnp.float32)])
    logit, lab = _run(x2, s1, scal)
    return logit.reshape(B, 1), (lab != 0).reshape(B, 1)


# speedup vs baseline: 298.1454x; 1.3146x over previous
import functools
import jax
import jax.numpy as jnp
from jax import lax
from jax.experimental import pallas as pl
from jax.experimental.pallas import tpu as pltpu
from jax.experimental.pallas import tpu_sc as plsc

B = 16384
L = 200
NCORES = 2
NSUB = 16
NW = NCORES * NSUB
ROWS_W = B // NW
CHUNK_ROWS = 64
NCHUNK = ROWS_W // CHUNK_ROWS
CHUNK_ELEMS = CHUNK_ROWS * L
VOCAB1 = 1000001
SPAD = 1024000
SLICE = SPAD // NSUB


UNROLL = 8


def _sc_body(x_hbm, s_hbm, scal_hbm, logit_hbm, label_hbm,
             idx_a, idx_b, val_a, val_b, out_f, out_i, scal_v, table_sp,
             sem_a, sem_b):
    c = lax.axis_index("c")
    s = lax.axis_index("s")
    wid = s * NCORES + c

    for j in range(SLICE // CHUNK_ELEMS):
        o = s * SLICE + j * CHUNK_ELEMS
        pltpu.sync_copy(s_hbm.at[pl.ds(o, CHUNK_ELEMS)], val_a)
        pltpu.sync_copy(val_a, table_sp.at[pl.ds(o, CHUNK_ELEMS)])

    pltpu.sync_copy(scal_hbm, scal_v)
    plsc.subcore_barrier()

    sv = scal_v[...]
    w_s = sv[0]
    b_s = sv[1]
    t_s = sv[2]

    lane = jax.lax.iota(jnp.int32, 16)
    bufs = ((idx_a, val_a, sem_a), (idx_b, val_b, sem_b))
    e_base = wid * ROWS_W * L

    def start(k):
        iv, vv, sem = bufs[k % 2]
        pltpu.sync_copy(x_hbm.at[pl.ds(e_base + k * CHUNK_ELEMS, CHUNK_ELEMS)],
                        iv)
        return pltpu.async_copy(table_sp.at[iv], vv, sem)

    copies = {0: start(0)}
    for k in range(NCHUNK):
        vref = bufs[k % 2][1]
        if k + 1 < NCHUNK:
            copies[k + 1] = start(k + 1)
        copies.pop(k).wait()

        for g in range(CHUNK_ROWS // 16):
            offs0 = (lane + g * 16) * L

            def body(l, carry):
                offs, acc = carry
                for u in range(UNROLL):
                    acc = acc + plsc.load_gather(vref, [offs + u])
                return offs + UNROLL, acc

            _, acc = lax.fori_loop(
                0, L // UNROLL, body, (offs0, jnp.zeros((16,), jnp.float32)))

            logit = acc * w_s + b_s
            lab = jnp.where(logit >= t_s,
                            jnp.full((16,), 1, jnp.int32),
                            jnp.full((16,), 0, jnp.int32))
            o = k * CHUNK_ROWS + g * 16
            out_f[pl.ds(o, 16)] = logit
            out_i[pl.ds(o, 16)] = lab

    pltpu.sync_copy(out_f, logit_hbm.at[pl.ds(wid * ROWS_W, ROWS_W)])
    pltpu.sync_copy(out_i, label_hbm.at[pl.ds(wid * ROWS_W, ROWS_W)])


@jax.jit
def _run(x2, s1, scal):
    f = functools.partial(
        pl.kernel,
        out_type=(jax.ShapeDtypeStruct((B,), jnp.float32),
                  jax.ShapeDtypeStruct((B,), jnp.int32)),
        mesh=plsc.VectorSubcoreMesh(core_axis_name="c", subcore_axis_name="s"),
        compiler_params=pltpu.CompilerParams(needs_layout_passes=False),
        scratch_types=[
            pltpu.VMEM((CHUNK_ELEMS,), jnp.int32),
            pltpu.VMEM((CHUNK_ELEMS,), jnp.int32),
            pltpu.VMEM((CHUNK_ELEMS,), jnp.float32),
            pltpu.VMEM((CHUNK_ELEMS,), jnp.float32),
            pltpu.VMEM((ROWS_W,), jnp.float32),
            pltpu.VMEM((ROWS_W,), jnp.int32),
            pltpu.VMEM((16,), jnp.float32),
            pltpu.VMEM_SHARED((SPAD,), jnp.float32),
            pltpu.SemaphoreType.DMA,
            pltpu.SemaphoreType.DMA,
        ],
    )(_sc_body)
    return f(x2, s1, scal)


def kernel(x, S, w, b, thresh_t):
    x2 = x.reshape(-1)
    s1 = jnp.pad(S, ((0, SPAD - VOCAB1), (0, 0))).reshape(-1)
    scal = jnp.concatenate(
        [w, b, thresh_t.reshape(-1), jnp.zeros((13,), jnp.float32)])
    logit, lab = _run(x2, s1, scal)
    return logit.reshape(B, 1), (lab != 0).reshape(B, 1)


# speedup vs baseline: 309.2722x; 1.0373x over previous
import functools
import jax
import jax.numpy as jnp
from jax import lax
from jax.experimental import pallas as pl
from jax.experimental.pallas import tpu as pltpu
from jax.experimental.pallas import tpu_sc as plsc

B = 16384
L = 200
NCORES = 2
NSUB = 16
NW = NCORES * NSUB
ROWS_W = B // NW
CHUNK_ROWS = 64
NCHUNK = ROWS_W // CHUNK_ROWS
CHUNK_ELEMS = CHUNK_ROWS * L
VOCAB1 = 1000001
SPAD = 1024000
SLICE = SPAD // NSUB


UNROLL = 8


def _sc_body(x_hbm, s_hbm, scal_hbm, logit_hbm, label_hbm,
             idx_a, idx_b, val_a, val_b, out_f, out_i, scal_v, table_sp,
             sem_a, sem_b, sem_ia, sem_ib):
    c = lax.axis_index("c")
    s = lax.axis_index("s")
    wid = s * NCORES + c

    idxb = (idx_a, idx_b)
    valb = (val_a, val_b)
    sem_i = (sem_ia, sem_ib)
    sem_v = (sem_a, sem_b)
    e_base = wid * ROWS_W * L

    def idx_start(k):
        return pltpu.async_copy(
            x_hbm.at[pl.ds(e_base + k * CHUNK_ELEMS, CHUNK_ELEMS)],
            idxb[k % 2], sem_i[k % 2])

    icp = {0: idx_start(0), 1: idx_start(1)}

    NP = SLICE // CHUNK_ELEMS
    t_base = s * SLICE

    def stage_start(j):
        return pltpu.async_copy(
            s_hbm.at[pl.ds(t_base + j * CHUNK_ELEMS, CHUNK_ELEMS)],
            valb[j % 2], sem_v[j % 2])

    scp = {0: stage_start(0)}
    for j in range(NP):
        if j + 1 < NP:
            scp[j + 1] = stage_start(j + 1)
        scp.pop(j).wait()
        pltpu.sync_copy(valb[j % 2],
                        table_sp.at[pl.ds(t_base + j * CHUNK_ELEMS,
                                          CHUNK_ELEMS)])

    pltpu.sync_copy(scal_hbm, scal_v)
    plsc.subcore_barrier()

    sv = scal_v[...]
    w_s = sv[0]
    b_s = sv[1]
    t_s = sv[2]

    lane = jax.lax.iota(jnp.int32, 16)

    def gather_start(k):
        return pltpu.async_copy(table_sp.at[idxb[k % 2]], valb[k % 2],
                                sem_v[k % 2])

    icp.pop(0).wait()
    gcp = {0: gather_start(0)}
    for k in range(NCHUNK):
        vref = valb[k % 2]
        gcp.pop(k).wait()
        if k + 2 < NCHUNK:
            icp[k + 2] = idx_start(k + 2)
        if k + 1 < NCHUNK:
            icp.pop(k + 1).wait()
            gcp[k + 1] = gather_start(k + 1)

        for g in range(CHUNK_ROWS // 16):
            offs0 = (lane + g * 16) * L

            def body(l, carry):
                offs, acc = carry
                for u in range(UNROLL):
                    acc = acc + plsc.load_gather(vref, [offs + u])
                return offs + UNROLL, acc

            _, acc = lax.fori_loop(
                0, L // UNROLL, body, (offs0, jnp.zeros((16,), jnp.float32)))

            logit = acc * w_s + b_s
            lab = jnp.where(logit >= t_s,
                            jnp.full((16,), 1, jnp.int32),
                            jnp.full((16,), 0, jnp.int32))
            o = k * CHUNK_ROWS + g * 16
            out_f[pl.ds(o, 16)] = logit
            out_i[pl.ds(o, 16)] = lab

    pltpu.sync_copy(out_f, logit_hbm.at[pl.ds(wid * ROWS_W, ROWS_W)])
    pltpu.sync_copy(out_i, label_hbm.at[pl.ds(wid * ROWS_W, ROWS_W)])


@jax.jit
def _run(x2, s1, scal):
    f = functools.partial(
        pl.kernel,
        out_type=(jax.ShapeDtypeStruct((B,), jnp.float32),
                  jax.ShapeDtypeStruct((B,), jnp.int32)),
        mesh=plsc.VectorSubcoreMesh(core_axis_name="c", subcore_axis_name="s"),
        compiler_params=pltpu.CompilerParams(needs_layout_passes=False),
        scratch_types=[
            pltpu.VMEM((CHUNK_ELEMS,), jnp.int32),
            pltpu.VMEM((CHUNK_ELEMS,), jnp.int32),
            pltpu.VMEM((CHUNK_ELEMS,), jnp.float32),
            pltpu.VMEM((CHUNK_ELEMS,), jnp.float32),
            pltpu.VMEM((ROWS_W,), jnp.float32),
            pltpu.VMEM((ROWS_W,), jnp.int32),
            pltpu.VMEM((16,), jnp.float32),
            pltpu.VMEM_SHARED((SPAD,), jnp.float32),
            pltpu.SemaphoreType.DMA,
            pltpu.SemaphoreType.DMA,
            pltpu.SemaphoreType.DMA,
            pltpu.SemaphoreType.DMA,
        ],
    )(_sc_body)
    return f(x2, s1, scal)


def kernel(x, S, w, b, thresh_t):
    x2 = x.reshape(-1)
    s1 = jnp.pad(S, ((0, SPAD - VOCAB1), (0, 0))).reshape(-1)
    scal = jnp.concatenate(
        [w, b, thresh_t.reshape(-1), jnp.zeros((13,), jnp.float32)])
    logit, lab = _run(x2, s1, scal)
    return logit.reshape(B, 1), (lab != 0).reshape(B, 1)


# speedup vs baseline: 346.6061x; 1.1207x over previous
import functools
import jax
import jax.numpy as jnp
from jax import lax
from jax.experimental import pallas as pl
from jax.experimental.pallas import tpu as pltpu
from jax.experimental.pallas import tpu_sc as plsc

B = 16384
L = 200
NCORES = 2
NSUB = 16
NW = NCORES * NSUB
ROWS_W = B // NW
CHUNK_ROWS = 32
NCHUNK = ROWS_W // CHUNK_ROWS
CHUNK_ELEMS = CHUNK_ROWS * L
VOCAB1 = 1000001
SPAD = 1024000
SLICE = SPAD // NSUB
NPIECE = SLICE // CHUNK_ELEMS
UNROLL = 8


def _sc_body(x_hbm, s_hbm, scal_hbm, logit_hbm, label_hbm,
             slab_a, slab_b, idx_a, idx_b, val_a, val_b, out_f, out_i,
             scal_v, table_sp, sem_a, sem_b, sem_sa, sem_sb):
    c = lax.axis_index("c")
    s = lax.axis_index("s")
    wid = s * NCORES + c

    slabs = (slab_a, slab_b)
    idxb = (idx_a, idx_b)
    valb = (val_a, val_b)
    sem_v = (sem_a, sem_b)
    sem_s = (sem_sa, sem_sb)
    r_base = wid * ROWS_W

    def slab_start(k):
        return pltpu.async_copy(
            x_hbm.at[pl.ds(r_base + k * CHUNK_ROWS, CHUNK_ROWS), :],
            slabs[k % 2], sem_s[k % 2])

    def repack(k):
        sl = slabs[k % 2]
        ix = idxb[k % 2]

        def rbody(r, carry):
            for cb in range(12):
                ix[pl.ds(r * L + cb * 16, 16)] = sl[r, pl.ds(cb * 16, 16)]
            ix[pl.ds(r * L + L - 16, 16)] = sl[r, pl.ds(L - 16, 16)]
            return carry

        lax.fori_loop(0, CHUNK_ROWS, rbody, jnp.int32(0))

    def gather_start(k):
        return pltpu.async_copy(table_sp.at[idxb[k % 2]], valb[k % 2],
                                sem_v[k % 2])

    scp = {0: slab_start(0), 1: slab_start(1)}

    t_base = s * SLICE

    def stage_start(j):
        return pltpu.async_copy(
            s_hbm.at[pl.ds(t_base + j * CHUNK_ELEMS, CHUNK_ELEMS)],
            valb[j % 2], sem_v[j % 2])

    stp = {0: stage_start(0)}
    for j in range(NPIECE):
        if j + 1 < NPIECE:
            stp[j + 1] = stage_start(j + 1)
        stp.pop(j).wait()
        pltpu.sync_copy(valb[j % 2],
                        table_sp.at[pl.ds(t_base + j * CHUNK_ELEMS,
                                          CHUNK_ELEMS)])

    pltpu.sync_copy(scal_hbm, scal_v)
    plsc.subcore_barrier()

    sv = scal_v[...]
    w_s = sv[0]
    b_s = sv[1]
    t_s = sv[2]

    lane = jax.lax.iota(jnp.int32, 16)

    scp.pop(0).wait()
    repack(0)
    gcp = {0: gather_start(0)}
    for k in range(NCHUNK):
        vref = valb[k % 2]
        if k + 2 < NCHUNK:
            scp[k + 2] = slab_start(k + 2)
        if k + 1 < NCHUNK:
            scp.pop(k + 1).wait()
            repack(k + 1)
            gcp[k + 1] = gather_start(k + 1)
        gcp.pop(k).wait()

        for g in range(CHUNK_ROWS // 16):
            offs0 = (lane + g * 16) * L

            def body(l, carry):
                offs, acc = carry
                for u in range(UNROLL):
                    acc = acc + plsc.load_gather(vref, [offs + u])
                return offs + UNROLL, acc

            _, acc = lax.fori_loop(
                0, L // UNROLL, body, (offs0, jnp.zeros((16,), jnp.float32)))

            logit = acc * w_s + b_s
            lab = jnp.where(logit >= t_s,
                            jnp.full((16,), 1, jnp.int32),
                            jnp.full((16,), 0, jnp.int32))
            o = k * CHUNK_ROWS + g * 16
            out_f[pl.ds(o, 16)] = logit
            out_i[pl.ds(o, 16)] = lab

    pltpu.sync_copy(out_f, logit_hbm.at[pl.ds(wid * ROWS_W, ROWS_W)])
    pltpu.sync_copy(out_i, label_hbm.at[pl.ds(wid * ROWS_W, ROWS_W)])


@jax.jit
def _run(x, s1, scal):
    f = functools.partial(
        pl.kernel,
        out_type=(jax.ShapeDtypeStruct((B,), jnp.float32),
                  jax.ShapeDtypeStruct((B,), jnp.int32)),
        mesh=plsc.VectorSubcoreMesh(core_axis_name="c", subcore_axis_name="s"),
        compiler_params=pltpu.CompilerParams(
            needs_layout_passes=False, use_tc_tiling_on_sc=True),
        scratch_types=[
            pltpu.VMEM((CHUNK_ROWS, L), jnp.int32),
            pltpu.VMEM((CHUNK_ROWS, L), jnp.int32),
            pltpu.VMEM((CHUNK_ELEMS,), jnp.int32),
            pltpu.VMEM((CHUNK_ELEMS,), jnp.int32),
            pltpu.VMEM((CHUNK_ELEMS,), jnp.float32),
            pltpu.VMEM((CHUNK_ELEMS,), jnp.float32),
            pltpu.VMEM((ROWS_W,), jnp.float32),
            pltpu.VMEM((ROWS_W,), jnp.int32),
            pltpu.VMEM((16,), jnp.float32),
            pltpu.VMEM_SHARED((SPAD,), jnp.float32),
            pltpu.SemaphoreType.DMA,
            pltpu.SemaphoreType.DMA,
            pltpu.SemaphoreType.DMA,
            pltpu.SemaphoreType.DMA,
        ],
    )(_sc_body)
    return f(x, s1, scal)


def kernel(x, S, w, b, thresh_t):
    s1 = jnp.pad(S, ((0, SPAD - VOCAB1), (0, 0))).reshape(-1)
    scal = jnp.concatenate(
        [w, b, thresh_t.reshape(-1), jnp.zeros((13,), jnp.float32)])
    logit, lab = _run(x, s1, scal)
    return logit.reshape(B, 1), (lab != 0).reshape(B, 1)


# speedup vs baseline: 347.4912x; 1.0026x over previous
import functools
import jax
import jax.numpy as jnp
from jax import lax
from jax.experimental import pallas as pl
from jax.experimental.pallas import tpu as pltpu
from jax.experimental.pallas import tpu_sc as plsc

B = 16384
L = 200
NCORES = 2
NSUB = 16
NW = NCORES * NSUB
ROWS_W = B // NW
CHUNK_ROWS = 32
NCHUNK = ROWS_W // CHUNK_ROWS
CHUNK_ELEMS = CHUNK_ROWS * L
VOCAB1 = 1000001
SPAD = 1024000
SLICE = SPAD // NSUB
NPIECE = SLICE // CHUNK_ELEMS
UNROLL = 8


def _sc_body(x_hbm, s_hbm, scal_hbm, logit_hbm, label_hbm,
             slab_a, slab_b, idx_a, idx_b, val_a, val_b, out_f, out_i,
             scal_v, table_sp, sem_a, sem_b, sem_sa, sem_sb):
    c = lax.axis_index("c")
    s = lax.axis_index("s")
    wid = s * NCORES + c

    slabs = (slab_a, slab_b)
    idxb = (idx_a, idx_b)
    valb = (val_a, val_b)
    sem_v = (sem_a, sem_b)
    sem_s = (sem_sa, sem_sb)
    r_base = wid * ROWS_W

    def slab_start(k):
        return pltpu.async_copy(
            x_hbm.at[pl.ds(r_base + k * CHUNK_ROWS, CHUNK_ROWS), :],
            slabs[k % 2], sem_s[k % 2])

    def repack(k):
        sl = slabs[k % 2]
        ix = idxb[k % 2]

        def rbody(r, carry):
            for cb in range(12):
                ix[pl.ds(r * L + cb * 16, 16)] = sl[r, pl.ds(cb * 16, 16)]
            ix[pl.ds(r * L + L - 16, 16)] = sl[r, pl.ds(L - 16, 16)]
            return carry

        lax.fori_loop(0, CHUNK_ROWS, rbody, jnp.int32(0))

    def gather_start(k):
        return pltpu.async_copy(table_sp.at[idxb[k % 2]], valb[k % 2],
                                sem_v[k % 2])

    scp = {0: slab_start(0), 1: slab_start(1)}

    t_base = s * SLICE

    def stage_start(j):
        return pltpu.async_copy(
            s_hbm.at[pl.ds(t_base + j * CHUNK_ELEMS, CHUNK_ELEMS)],
            valb[j % 2], sem_v[j % 2])

    stp = {0: stage_start(0)}
    for j in range(NPIECE):
        if j + 1 < NPIECE:
            stp[j + 1] = stage_start(j + 1)
        stp.pop(j).wait()
        pltpu.sync_copy(valb[j % 2],
                        table_sp.at[pl.ds(t_base + j * CHUNK_ELEMS,
                                          CHUNK_ELEMS)])

    pltpu.sync_copy(scal_hbm, scal_v)
    plsc.subcore_barrier()

    sv = scal_v[...]
    w_s = sv[0]
    b_s = sv[1]
    t_s = sv[2]

    lane = jax.lax.iota(jnp.int32, 16)

    scp.pop(0).wait()
    repack(0)
    gcp = {0: gather_start(0)}
    for k in range(NCHUNK):
        vref = valb[k % 2]
        if k + 2 < NCHUNK:
            scp[k + 2] = slab_start(k + 2)
        if k + 1 < NCHUNK:
            scp.pop(k + 1).wait()
            repack(k + 1)
            gcp[k + 1] = gather_start(k + 1)
        gcp.pop(k).wait()

        for g in range(CHUNK_ROWS // 16):
            offs0 = (lane + g * 16) * L

            def body(l, carry):
                offs, acc = carry
                for u in range(UNROLL):
                    acc = acc + plsc.load_gather(vref, [offs + u])
                return offs + UNROLL, acc

            _, acc = lax.fori_loop(
                0, L // UNROLL, body, (offs0, jnp.zeros((16,), jnp.float32)))

            logit = acc * w_s + b_s
            lab = jnp.where(logit >= t_s,
                            jnp.full((16,), 1, jnp.int32),
                            jnp.full((16,), 0, jnp.int32))
            o = k * CHUNK_ROWS + g * 16
            out_f[pl.ds(o, 16)] = logit
            out_i[pl.ds(o, 16)] = lab

    pltpu.sync_copy(out_f, logit_hbm.at[pl.ds(wid * ROWS_W, ROWS_W)])
    pltpu.sync_copy(out_i, label_hbm.at[pl.ds(wid * ROWS_W, ROWS_W)])


@jax.jit
def _run(x, s1, scal):
    f = functools.partial(
        pl.kernel,
        out_type=(jax.ShapeDtypeStruct((B,), jnp.float32),
                  jax.ShapeDtypeStruct((B,), jnp.int32)),
        mesh=plsc.VectorSubcoreMesh(core_axis_name="c", subcore_axis_name="s"),
        compiler_params=pltpu.CompilerParams(
            needs_layout_passes=False, use_tc_tiling_on_sc=True),
        scratch_types=[
            pltpu.VMEM((CHUNK_ROWS, L), jnp.int32),
            pltpu.VMEM((CHUNK_ROWS, L), jnp.int32),
            pltpu.VMEM((CHUNK_ELEMS,), jnp.int32),
            pltpu.VMEM((CHUNK_ELEMS,), jnp.int32),
            pltpu.VMEM((CHUNK_ELEMS,), jnp.float32),
            pltpu.VMEM((CHUNK_ELEMS,), jnp.float32),
            pltpu.VMEM((ROWS_W,), jnp.float32),
            pltpu.VMEM((ROWS_W,), jnp.int32),
            pltpu.VMEM((16,), jnp.float32),
            pltpu.VMEM_SHARED((SPAD,), jnp.float32),
            pltpu.SemaphoreType.DMA,
            pltpu.SemaphoreType.DMA,
            pltpu.SemaphoreType.DMA,
            pltpu.SemaphoreType.DMA,
        ],
    )(_sc_body)
    return f(x, s1, scal)


def kernel(x, S, w, b, thresh_t):
    s1 = jnp.concatenate(
        [S, jnp.zeros((SPAD - VOCAB1, 1), jnp.float32)], axis=0).reshape(-1)
    scal = jnp.concatenate(
        [w, b, thresh_t.reshape(-1), jnp.zeros((13,), jnp.float32)])
    logit, lab = _run(x, s1, scal)
    return logit.reshape(B, 1), (lab != 0).reshape(B, 1)
